# baseline (speedup 1.0000x reference)
import functools

import jax
import jax.numpy as jnp
from jax import lax
from jax.experimental import pallas as pl
from jax.experimental.pallas import tpu as pltpu
from jax.experimental.pallas import tpu_sc as plsc

_VOCAB = 1000000
_D = 64
_B = 4096
_T = 200
_N = _B * _T
_NC = 2
_NS = 16
_NW = _NC * _NS
_PER_W = _N // _NW
_CH = 512
_NCH = _PER_W // _CH
_SCALE = 8.0

_mesh = plsc.VectorSubcoreMesh(core_axis_name="c", subcore_axis_name="s")


@functools.partial(
    pl.kernel,
    mesh=_mesh,
    out_type=jax.ShapeDtypeStruct((_N, _D), jnp.float32),
    scratch_types=[
        pltpu.VMEM((_PER_W,), jnp.int32),
        pltpu.VMEM((_CH, _D), jnp.float32),
        pltpu.SemaphoreType.DMA,
    ],
    compiler_params=pltpu.CompilerParams(use_tc_tiling_on_sc=False),
)
def _embed(idx_hbm, table_hbm, out_hbm, idx_v, rows_v, sem):
    wid = lax.axis_index("s") * _NC + lax.axis_index("c")
    base = wid * _PER_W
    pltpu.sync_copy(idx_hbm.at[pl.ds(base, _PER_W)], idx_v)

    def chunk(i, _):
        pltpu.async_copy(
            table_hbm.at[idx_v.at[pl.ds(i * _CH, _CH)]], rows_v, sem
        ).wait()

        def scale_row(r, _):
            for k in range(_D // 16):
                sl = (r, pl.ds(k * 16, 16))
                rows_v[sl] = rows_v[sl] * _SCALE
            return _

        lax.fori_loop(0, _CH, scale_row, 0, unroll=False)
        pltpu.sync_copy(rows_v, out_hbm.at[pl.ds(base + i * _CH, _CH)])
        return _

    lax.fori_loop(0, _NCH, chunk, 0, unroll=False)


def kernel(x, table):
    idx = x.reshape(_N)
    out = _embed(idx, table)
    return out.reshape(_B, _T, _D)


# speedup vs baseline: 1.1186x; 1.1186x over previous
import functools

import jax
import jax.numpy as jnp
from jax import lax
from jax.experimental import pallas as pl
from jax.experimental.pallas import tpu as pltpu
from jax.experimental.pallas import tpu_sc as plsc

_VOCAB = 1000000
_D = 64
_B = 4096
_T = 200
_N = _B * _T
_NC = 2
_NS = 16
_NW = _NC * _NS
_PER_W = _N // _NW
_CH = 320
_NCH = _PER_W // _CH
_NB = 4
_SCALE = 8.0

_mesh = plsc.VectorSubcoreMesh(core_axis_name="c", subcore_axis_name="s")


@functools.partial(
    pl.kernel,
    mesh=_mesh,
    out_type=jax.ShapeDtypeStruct((_N, _D), jnp.float32),
    scratch_types=(
        [pltpu.VMEM((_PER_W,), jnp.int32)]
        + [pltpu.VMEM((_CH, _D), jnp.float32)] * _NB
        + [pltpu.SemaphoreType.DMA] * (2 * _NB)
    ),
    compiler_params=pltpu.CompilerParams(use_tc_tiling_on_sc=False),
)
def _embed(idx_hbm, table_hbm, out_hbm, idx_v, *scratch):
    bufs = scratch[:_NB]
    gsems = scratch[_NB:2 * _NB]
    ssems = scratch[2 * _NB:]

    wid = lax.axis_index("s") * _NC + lax.axis_index("c")
    base = wid * _PER_W
    pltpu.sync_copy(idx_hbm.at[pl.ds(base, _PER_W)], idx_v)

    def gather_desc(c, b):
        src = table_hbm.at[idx_v.at[pl.ds(c * _CH, _CH)]]
        return pltpu.make_async_copy(src, bufs[b], gsems[b])

    def store_desc(c, b):
        dst = out_hbm.at[pl.ds(base + c * _CH, _CH)]
        return pltpu.make_async_copy(bufs[b], dst, ssems[b])

    def scale(b):
        buf = bufs[b]

        def row(r, carry):
            for k in range(_D // 16):
                sl = (r, pl.ds(k * 16, 16))
                buf[sl] = buf[sl] * _SCALE
            return carry

        lax.fori_loop(0, _CH, row, 0, unroll=8)

    def head(c, b):
        gather_desc(c, b).wait()
        scale(b)
        store_desc(c, b).start()

    def tail(c, b):
        wait_store_of = c - 2
        nb = (b + 2) % _NB
        store_desc(wait_store_of, nb).wait()
        gather_desc(c + 2, nb).start()

    gather_desc(0, 0).start()
    gather_desc(1, 1).start()
    head(0, 0)
    gather_desc(2, 2).start()
    head(1, 1)
    gather_desc(3, 3).start()
    head(2, 2)
    tail(2, 2)
    head(3, 3)
    tail(3, 3)

    def step(o, carry):
        c0 = o * _NB
        for u in range(_NB):
            head(c0 + u, u)
            tail(c0 + u, u)
        return carry

    lax.fori_loop(1, _NCH // _NB - 1, step, 0, unroll=False)

    head(_NCH - 4, 0)
    tail(_NCH - 4, 0)
    head(_NCH - 3, 1)
    tail(_NCH - 3, 1)
    head(_NCH - 2, 2)
    head(_NCH - 1, 3)
    store_desc(_NCH - 4, 0).wait()
    store_desc(_NCH - 3, 1).wait()
    store_desc(_NCH - 2, 2).wait()
    store_desc(_NCH - 1, 3).wait()


def kernel(x, table):
    idx = x.reshape(_N)
    out = _embed(idx, table)
    return out.reshape(_B, _T, _D)
